# baseline (speedup 1.0000x reference)
import functools

import jax
import jax.numpy as jnp
from jax import lax
from jax.experimental import pallas as pl

D_MODEL = 1024
D_FF = 4096
H = 16
DH = 64
N_BUCKETS = 64
N_HASHES = 4
CHUNK = 64
GRP = 8
TOK_BLK = 512
TOK_BLK_C = 256


def _proj_body(x_ref, g_ref, b_ref, wqk_ref, wv_ref, rot_ref,
               qk_ref, v_ref, rt_ref):
    x = x_ref[...]
    m = jnp.mean(x, axis=-1, keepdims=True)
    xc = x - m
    var = jnp.mean(xc * xc, axis=-1, keepdims=True)
    xn = xc * lax.rsqrt(var + 1e-5) * g_ref[...] + b_ref[...]
    qk = jnp.dot(xn, wqk_ref[...], preferred_element_type=jnp.float32)
    qk_ref[...] = qk
    v_ref[...] = jnp.dot(xn, wv_ref[...], preferred_element_type=jnp.float32)
    rt_ref[...] = jnp.dot(qk, rot_ref[...], preferred_element_type=jnp.float32)


def _attn_body(q_ref, qp_ref, v_ref, vp_ref, br_ref, bp_ref, pr_ref, pp_ref,
               o_ref, lse_ref):
    qfull = q_ref[0]
    qprev = qp_ref[0]
    vfull = v_ref[0]
    vprev = vp_ref[0]
    brow = br_ref[0, 0]
    brow_p = bp_ref[0, 0]
    prow = pr_ref[0, 0]
    prow_p = pp_ref[0, 0]
    for j in range(GRP):
        lo = j * CHUNK
        qj = qfull[lo:lo + CHUNK, :DH]
        bq = qfull[lo:lo + CHUNK, DH:DH + 1]
        pq = qfull[lo:lo + CHUNK, DH + 1:DH + 2]
        if j == 0:
            kprev = qprev[(GRP - 1) * CHUNK:, :DH]
            vprevj = vprev[(GRP - 1) * CHUNK:]
            b_prev = brow_p[GRP - 1:GRP]
            p_prev = prow_p[GRP - 1:GRP]
        else:
            kprev = qfull[lo - CHUNK:lo, :DH]
            vprevj = vfull[lo - CHUNK:lo]
            b_prev = brow[j - 1:j]
            p_prev = prow[j - 1:j]
        kcat = jnp.concatenate([kprev, qfull[lo:lo + CHUNK, :DH]], axis=0)
        vcat = jnp.concatenate([vprevj, vfull[lo:lo + CHUNK]], axis=0)
        knorm = kcat * (1.0 / (jnp.sqrt(
            jnp.sum(kcat * kcat, axis=-1, keepdims=True)) + 1e-6))
        dots = lax.dot_general(qj, knorm, (((1,), (1,)), ((), ())),
                               preferred_element_type=jnp.float32)
        dots = dots * (1.0 / (float(DH) ** 0.5))
        b_e = jnp.concatenate([b_prev, brow[j:j + 1]], axis=1)
        p_e = jnp.concatenate([p_prev, prow[j:j + 1]], axis=1)
        dots = jnp.where(bq == b_e, dots, -1e9)
        dots = jnp.where(pq == p_e, -1e5, dots)
        m = jnp.max(dots, axis=-1, keepdims=True)
        e = jnp.exp(dots - m)
        s = jnp.sum(e, axis=-1, keepdims=True)
        o = jnp.dot(e, vcat, preferred_element_type=jnp.float32) / s
        o_ref[0, lo:lo + CHUNK, :] = o
        lse_ref[0, lo:lo + CHUNK, :] = m + jnp.log(s)


def _out_ff_body(o_ref, x1_ref, x2_ref, wo_ref, g_ref, b_ref,
                 w1_ref, b1_ref, w2_ref, b2_ref, y1_ref, y2_ref):
    y1 = x1_ref[...] + jnp.dot(o_ref[...], wo_ref[...],
                               preferred_element_type=jnp.float32)
    y1_ref[...] = y1
    m = jnp.mean(y1, axis=-1, keepdims=True)
    xc = y1 - m
    var = jnp.mean(xc * xc, axis=-1, keepdims=True)
    t = xc * lax.rsqrt(var + 1e-5) * g_ref[...] + b_ref[...]
    h = jax.nn.gelu(jnp.dot(t, w1_ref[...],
                            preferred_element_type=jnp.float32) + b1_ref[...])
    y2_ref[...] = x2_ref[...] + jnp.dot(h, w2_ref[...],
                                        preferred_element_type=jnp.float32) + b2_ref[...]


def _build_rotmat():
    rot = jax.random.normal(jax.random.key(42),
                            (N_HASHES, DH, N_BUCKETS // 2), dtype=jnp.float32)
    eye = jnp.eye(H, dtype=jnp.float32)
    blk = jnp.einsum('gh,rdn->rgdhn', eye, rot)
    return blk.transpose(1, 2, 0, 3, 4).reshape(D_MODEL, N_HASHES * H * 32)


def kernel(x1, x2, Wqk, Wv, Wo, W1, b1, W2, b2, ln1_g, ln1_b, ln2_g, ln2_b):
    B, S, _ = x1.shape
    nc = S // CHUNK
    ng = nc // GRP
    T = B * S
    nblk = T // TOK_BLK
    inst = N_HASHES * B * H

    rotmat = _build_rotmat()
    x2f = x2.reshape(T, D_MODEL)

    row = lambda a: a.reshape(1, -1)
    full = lambda r, c: pl.BlockSpec((r, c), lambda i: (0, 0))
    qk, v, rt = pl.pallas_call(
        _proj_body,
        grid=(nblk,),
        in_specs=[
            pl.BlockSpec((TOK_BLK, D_MODEL), lambda i: (i, 0)),
            full(1, D_MODEL), full(1, D_MODEL),
            full(D_MODEL, D_MODEL), full(D_MODEL, D_MODEL),
            full(D_MODEL, N_HASHES * H * 32),
        ],
        out_specs=[
            pl.BlockSpec((TOK_BLK, D_MODEL), lambda i: (i, 0)),
            pl.BlockSpec((TOK_BLK, D_MODEL), lambda i: (i, 0)),
            pl.BlockSpec((TOK_BLK, N_HASHES * H * 32), lambda i: (i, 0)),
        ],
        out_shape=[
            jax.ShapeDtypeStruct((T, D_MODEL), jnp.float32),
            jax.ShapeDtypeStruct((T, D_MODEL), jnp.float32),
            jax.ShapeDtypeStruct((T, N_HASHES * H * 32), jnp.float32),
        ],
    )(x2f, row(ln1_g), row(ln1_b), Wqk, Wv, rotmat)

    rt = rt.reshape(B, S, N_HASHES, H, 32)
    rt = jnp.concatenate([rt, -rt], axis=-1)
    buckets = jnp.argmax(rt, axis=-1).astype(jnp.int32)
    buckets = buckets.transpose(2, 0, 3, 1)
    pos = jnp.arange(S, dtype=jnp.int32)
    skey = buckets * S + pos[None, None, None, :]
    perm = jnp.argsort(skey, axis=-1)
    inv = jnp.argsort(perm, axis=-1)
    sb = jnp.take_along_axis(buckets, perm, axis=-1)

    qkh = qk.reshape(B, S, H, DH).transpose(0, 2, 1, 3)
    vh = v.reshape(B, S, H, DH).transpose(0, 2, 1, 3)
    sqk = jnp.take_along_axis(qkh[None], perm[..., None], axis=3)
    sv = jnp.take_along_axis(vh[None], perm[..., None], axis=3)

    sbf = sb.astype(jnp.float32)
    spf = perm.astype(jnp.float32)
    a_q = jnp.concatenate([sqk, sbf[..., None], spf[..., None]], axis=-1)
    a_q = a_q.reshape(inst, S, DH + 2)
    a_v = sv.reshape(inst, S, DH)
    b_row = sbf.reshape(inst, ng, 1, GRP * CHUNK).reshape(inst, ng, GRP, CHUNK)
    p_row = spf.reshape(inst, ng, GRP, CHUNK)

    o_s, lse_s = pl.pallas_call(
        _attn_body,
        grid=(inst, ng),
        in_specs=[
            pl.BlockSpec((1, GRP * CHUNK, DH + 2), lambda i, g: (i, g, 0)),
            pl.BlockSpec((1, GRP * CHUNK, DH + 2),
                         lambda i, g: (i, (g + ng - 1) % ng, 0)),
            pl.BlockSpec((1, GRP * CHUNK, DH), lambda i, g: (i, g, 0)),
            pl.BlockSpec((1, GRP * CHUNK, DH),
                         lambda i, g: (i, (g + ng - 1) % ng, 0)),
            pl.BlockSpec((1, 1, GRP, CHUNK), lambda i, g: (i, g, 0, 0)),
            pl.BlockSpec((1, 1, GRP, CHUNK),
                         lambda i, g: (i, (g + ng - 1) % ng, 0, 0)),
            pl.BlockSpec((1, 1, GRP, CHUNK), lambda i, g: (i, g, 0, 0)),
            pl.BlockSpec((1, 1, GRP, CHUNK),
                         lambda i, g: (i, (g + ng - 1) % ng, 0, 0)),
        ],
        out_specs=[
            pl.BlockSpec((1, GRP * CHUNK, DH), lambda i, g: (i, g, 0)),
            pl.BlockSpec((1, GRP * CHUNK, 1), lambda i, g: (i, g, 0)),
        ],
        out_shape=[
            jax.ShapeDtypeStruct((inst, S, DH), jnp.float32),
            jax.ShapeDtypeStruct((inst, S, 1), jnp.float32),
        ],
    )(a_q, a_q, a_v, a_v, b_row, b_row, p_row, p_row)

    o_s = o_s.reshape(N_HASHES, B, H, S, DH)
    lse_s = lse_s.reshape(N_HASHES, B, H, S)
    o_all = jnp.take_along_axis(o_s, inv[..., None], axis=3)
    lse_all = jnp.take_along_axis(lse_s, inv, axis=3)
    w = jax.nn.softmax(lse_all, axis=0)[..., None]
    o_comb = jnp.sum(o_all * w, axis=0)
    o_comb = o_comb.transpose(0, 2, 1, 3).reshape(T, D_MODEL)

    y1, y2 = pl.pallas_call(
        _out_ff_body,
        grid=(T // TOK_BLK_C,),
        in_specs=[
            pl.BlockSpec((TOK_BLK_C, D_MODEL), lambda i: (i, 0)),
            pl.BlockSpec((TOK_BLK_C, D_MODEL), lambda i: (i, 0)),
            pl.BlockSpec((TOK_BLK_C, D_MODEL), lambda i: (i, 0)),
            full(D_MODEL, D_MODEL),
            full(1, D_MODEL), full(1, D_MODEL),
            full(D_MODEL, D_FF), full(1, D_FF),
            full(D_FF, D_MODEL), full(1, D_MODEL),
        ],
        out_specs=[
            pl.BlockSpec((TOK_BLK_C, D_MODEL), lambda i: (i, 0)),
            pl.BlockSpec((TOK_BLK_C, D_MODEL), lambda i: (i, 0)),
        ],
        out_shape=[
            jax.ShapeDtypeStruct((T, D_MODEL), jnp.float32),
            jax.ShapeDtypeStruct((T, D_MODEL), jnp.float32),
        ],
    )(o_comb, x1.reshape(T, D_MODEL), x2f, Wo, row(ln2_g), row(ln2_b),
      W1, row(b1), W2, row(b2))

    return (y1.reshape(B, S, D_MODEL), y2.reshape(B, S, D_MODEL))
